# baseline (speedup 1.0000x reference)
import jax
import jax.numpy as jnp
from jax.experimental import pallas as pl
from jax.experimental.pallas import tpu as pltpu
from functools import partial


def _gcab_kernel(ei_ref, x_ref, W0_hbm, as0_ref, ad0_ref, b0_ref,
                 W1_hbm, as1_ref, ad1_ref, b1_ref, Wm_hbm, bm_ref, Wg_ref,
                 bg_ref, out_ref, w0_ref, w1_ref, wm_ref, sem0, sem1, semm, *, n):
    cp0 = pltpu.make_async_copy(W0_hbm, w0_ref, sem0)
    cp1 = pltpu.make_async_copy(W1_hbm, w1_ref, sem1)
    cpm = pltpu.make_async_copy(Wm_hbm, wm_ref, semm)
    cp0.start()
    cp1.start()
    cpm.start()

    e = ei_ref.shape[1]
    rows = jax.lax.broadcasted_iota(jnp.int32, (n, e), 0)
    S = (ei_ref[0:1, :] == rows).astype(jnp.bfloat16)
    D = (ei_ref[1:2, :] == rows).astype(jnp.bfloat16)
    A = jax.lax.dot_general(D, S, (((1,), (1,)), ((), ())),
                            preferred_element_type=jnp.float32)
    ri = jax.lax.broadcasted_iota(jnp.int32, (n, n), 0)
    ci = jax.lax.broadcasted_iota(jnp.int32, (n, n), 1)
    A = A + (ri == ci).astype(jnp.float32)

    ones_row = jnp.ones((1, n), jnp.float32)
    ones_col = jnp.ones((n, 1), jnp.float32)

    def gat(h_in, W, a_s_ref, a_d_ref, b_ref):
        h = jnp.dot(h_in, W, preferred_element_type=jnp.float32)
        a_src_row = jax.lax.dot_general(a_s_ref[:], h, (((1,), (1,)), ((), ())),
                                        preferred_element_type=jnp.float32)
        a_dst_col = jax.lax.dot_general(h, a_d_ref[:], (((1,), (1,)), ((), ())),
                                        preferred_element_type=jnp.float32)
        lhs = jnp.concatenate([a_dst_col, ones_col], axis=1)
        rhs = jnp.concatenate([ones_row, a_src_row], axis=0)
        alpha = jnp.dot(lhs, rhs, preferred_element_type=jnp.float32)
        alpha = jnp.maximum(alpha, 0.2 * alpha)
        ex = jnp.exp(alpha) * A
        den = jnp.dot(ex, ones_col, preferred_element_type=jnp.float32)
        agg = jnp.dot(ex, h, preferred_element_type=jnp.float32)
        return agg * (1.0 / (den + 1e-16)) + b_ref[:]

    cp0.wait()
    h = gat(x_ref[:], w0_ref[:], as0_ref, ad0_ref, b0_ref)
    res = h
    cp1.wait()
    h = gat(h, w1_ref[:], as1_ref, ad1_ref, b1_ref)
    h = jnp.maximum(h + res, 0.0)

    avg = jnp.mean(h, axis=0, keepdims=True)
    mx = jnp.max(h, axis=0, keepdims=True)
    cpm.wait()
    za = jax.lax.dot_general(avg, wm_ref[:], (((1,), (1,)), ((), ())),
                             preferred_element_type=jnp.float32) + bm_ref[:]
    zm = jax.lax.dot_general(mx, wm_ref[:], (((1,), (1,)), ((), ())),
                             preferred_element_type=jnp.float32) + bm_ref[:]
    mch = jax.nn.sigmoid(jnp.maximum(za, 0.0) + jnp.maximum(zm, 0.0))
    h = h * mch

    hmax = jnp.max(h, axis=1, keepdims=True)
    deg = jnp.sum(A, axis=1, keepdims=True)
    dinv = jnp.where(deg > 0.0, jax.lax.rsqrt(deg), 0.0)
    g = hmax * Wg_ref[0, 0]
    agg = jnp.dot(A, dinv * g, preferred_element_type=jnp.float32)
    mno = jax.nn.sigmoid(dinv * agg + bg_ref[0, 0])
    out_ref[:] = h * mno


@jax.jit
def kernel(x, edge_index, W0, att_src0, att_dst0, b0, W1, att_src1, att_dst1,
           b1, Wm, bm, Wg, bg):
    n, din = x.shape
    dout = W0.shape[1]
    vmem = pl.BlockSpec(memory_space=pltpu.MemorySpace.VMEM)
    hbm = pl.BlockSpec(memory_space=pltpu.MemorySpace.HBM)
    f = pl.pallas_call(
        partial(_gcab_kernel, n=n),
        out_shape=jax.ShapeDtypeStruct((n, dout), jnp.float32),
        in_specs=[vmem, vmem, hbm, vmem, vmem, vmem, hbm, vmem, vmem, vmem,
                  hbm, vmem, vmem, vmem],
        scratch_shapes=[
            pltpu.VMEM((din, dout), jnp.float32),
            pltpu.VMEM((din, dout), jnp.float32),
            pltpu.VMEM((dout, dout), jnp.float32),
            pltpu.SemaphoreType.DMA,
            pltpu.SemaphoreType.DMA,
            pltpu.SemaphoreType.DMA,
        ],
    )
    return f(edge_index, x, W0, att_src0.reshape(1, dout), att_dst0.reshape(1, dout),
             b0.reshape(1, dout), W1, att_src1.reshape(1, dout),
             att_dst1.reshape(1, dout), b1.reshape(1, dout), Wm,
             bm.reshape(1, dout), Wg, bg.reshape(1, 1))


# speedup vs baseline: 1.0743x; 1.0743x over previous
import jax
import jax.numpy as jnp
from jax.experimental import pallas as pl
from jax.experimental.pallas import tpu as pltpu
from functools import partial


def _gcab_kernel(ei_ref, x_ref, W0_ref, as0_ref, ad0_ref, b0_ref,
                 W1_hbm, as1_ref, ad1_ref, b1_ref, Wm_hbm, bm_ref, Wg_ref,
                 bg_ref, out_ref, *, n):
    w1_ref = W1_hbm
    wm_ref = Wm_hbm

    e = ei_ref.shape[1]
    rows = jax.lax.broadcasted_iota(jnp.int32, (n, e), 0)
    S = (ei_ref[0:1, :] == rows).astype(jnp.bfloat16)
    D = (ei_ref[1:2, :] == rows).astype(jnp.bfloat16)
    A = jax.lax.dot_general(D, S, (((1,), (1,)), ((), ())),
                            preferred_element_type=jnp.float32)
    ri = jax.lax.broadcasted_iota(jnp.int32, (n, n), 0)
    ci = jax.lax.broadcasted_iota(jnp.int32, (n, n), 1)
    A = A + (ri == ci).astype(jnp.float32)

    ones_row = jnp.ones((1, n), jnp.float32)
    ones_col = jnp.ones((n, 1), jnp.float32)

    def gat(h_in, W, a_s_ref, a_d_ref, b_ref):
        h = jnp.dot(h_in, W, preferred_element_type=jnp.float32)
        a_src_row = jax.lax.dot_general(a_s_ref[:], h, (((1,), (1,)), ((), ())),
                                        preferred_element_type=jnp.float32)
        a_dst_col = jax.lax.dot_general(h, a_d_ref[:], (((1,), (1,)), ((), ())),
                                        preferred_element_type=jnp.float32)
        lhs = jnp.concatenate([a_dst_col, ones_col], axis=1)
        rhs = jnp.concatenate([ones_row, a_src_row], axis=0)
        alpha = jnp.dot(lhs, rhs, preferred_element_type=jnp.float32)
        alpha = jnp.maximum(alpha, 0.2 * alpha)
        ex = jnp.exp(alpha) * A
        den = jnp.dot(ex, ones_col, preferred_element_type=jnp.float32)
        agg = jnp.dot(ex, h, preferred_element_type=jnp.float32)
        return agg * (1.0 / (den + 1e-16)) + b_ref[:]

    h = gat(x_ref[:], W0_ref[:], as0_ref, ad0_ref, b0_ref)
    res = h
    h = gat(h, w1_ref[:], as1_ref, ad1_ref, b1_ref)
    h = jnp.maximum(h + res, 0.0)

    avg = jnp.mean(h, axis=0, keepdims=True)
    mx = jnp.max(h, axis=0, keepdims=True)
    za = jax.lax.dot_general(avg, wm_ref[:], (((1,), (1,)), ((), ())),
                             preferred_element_type=jnp.float32) + bm_ref[:]
    zm = jax.lax.dot_general(mx, wm_ref[:], (((1,), (1,)), ((), ())),
                             preferred_element_type=jnp.float32) + bm_ref[:]
    mch = jax.nn.sigmoid(jnp.maximum(za, 0.0) + jnp.maximum(zm, 0.0))
    h = h * mch

    hmax = jnp.max(h, axis=1, keepdims=True)
    deg = jnp.sum(A, axis=1, keepdims=True)
    dinv = jnp.where(deg > 0.0, jax.lax.rsqrt(deg), 0.0)
    g = hmax * Wg_ref[0, 0]
    agg = jnp.dot(A, dinv * g, preferred_element_type=jnp.float32)
    mno = jax.nn.sigmoid(dinv * agg + bg_ref[0, 0])
    out_ref[:] = h * mno


@jax.jit
def kernel(x, edge_index, W0, att_src0, att_dst0, b0, W1, att_src1, att_dst1,
           b1, Wm, bm, Wg, bg):
    n, din = x.shape
    dout = W0.shape[1]
    vmem = pl.BlockSpec(memory_space=pltpu.MemorySpace.VMEM)
    hbm = pl.BlockSpec(memory_space=pltpu.MemorySpace.HBM)
    f = pl.pallas_call(
        partial(_gcab_kernel, n=n),
        out_shape=jax.ShapeDtypeStruct((n, dout), jnp.float32),
        in_specs=[vmem] * 14,
    )
    return f(edge_index, x, W0, att_src0.reshape(1, dout), att_dst0.reshape(1, dout),
             b0.reshape(1, dout), W1, att_src1.reshape(1, dout),
             att_dst1.reshape(1, dout), b1.reshape(1, dout), Wm,
             bm.reshape(1, dout), Wg, bg.reshape(1, 1))
